# baseline (speedup 1.0000x reference)
import functools

import jax
import jax.numpy as jnp
from jax import lax
from jax.experimental import pallas as pl
from jax.experimental.pallas import tpu as pltpu
from jax.experimental.pallas import tpu_sc as plsc

B = 16384
EMB = 128
NC = 2
NS = 16
NW = NC * NS
BPW = B // NW
CHUNK = 256
NCHUNK = BPW // CHUNK


def _sc_body(users_hbm, items_hbm, negs_hbm, umem_hbm, imem_hbm, vw_hbm,
             pos_hbm, neg_hbm,
             uidx_v, iidx_v, nidx_v, urows_v, irows_v, nrows_v,
             v_v, pos_v, neg_v, sem):
    wid = lax.axis_index("s") * NC + lax.axis_index("c")
    base = wid * BPW
    pltpu.sync_copy(vw_hbm, v_v)

    for c in range(NCHUNK):
        cbase = base + c * CHUNK
        pltpu.sync_copy(users_hbm.at[pl.ds(cbase, CHUNK)], uidx_v)
        pltpu.sync_copy(items_hbm.at[pl.ds(cbase, CHUNK)], iidx_v)
        pltpu.sync_copy(negs_hbm.at[pl.ds(cbase, CHUNK)], nidx_v)
        cp_u = pltpu.async_copy(umem_hbm.at[uidx_v], urows_v, sem)
        cp_i = pltpu.async_copy(imem_hbm.at[iidx_v], irows_v, sem)
        cp_n = pltpu.async_copy(imem_hbm.at[nidx_v], nrows_v, sem)
        cp_u.wait()
        cp_i.wait()
        cp_n.wait()

        vslices = [v_v[0, pl.ds(16 * j, 16)] for j in range(EMB // 16)]

        def group_body(g, carry):
            r0 = g * 16
            rows = r0 + lax.iota(jnp.int32, 16)
            ap = jnp.zeros((16,), jnp.float32)
            an = jnp.zeros((16,), jnp.float32)
            for k in range(EMB):
                cols = jnp.full((16,), k, jnp.int32)
                u = plsc.load_gather(urows_v, [rows, cols])
                i = plsc.load_gather(irows_v, [rows, cols])
                n = plsc.load_gather(nrows_v, [rows, cols])
                vb = vslices[k // 16].at[
                    jnp.full((16,), k % 16, jnp.int32)
                ].get(mode="promise_in_bounds")
                t = u * vb
                ap = ap + t * i
                an = an + t * n
            pos_v[pl.ds(r0, 16)] = jnp.maximum(ap, 0.0)
            neg_v[pl.ds(r0, 16)] = jnp.maximum(an, 0.0)
            return carry

        lax.fori_loop(0, CHUNK // 16, group_body, 0)
        pltpu.sync_copy(pos_v, pos_hbm.at[pl.ds(cbase, CHUNK)])
        pltpu.sync_copy(neg_v, neg_hbm.at[pl.ds(cbase, CHUNK)])


@jax.jit
def _run(users, items, negs, umem, imem, vw):
    f = pl.kernel(
        _sc_body,
        out_type=(
            jax.ShapeDtypeStruct((B,), jnp.float32),
            jax.ShapeDtypeStruct((B,), jnp.float32),
        ),
        mesh=plsc.VectorSubcoreMesh(core_axis_name="c", subcore_axis_name="s"),
        compiler_params=pltpu.CompilerParams(use_tc_tiling_on_sc=False,
                                             needs_layout_passes=False),
        scratch_types=[
            pltpu.VMEM((CHUNK,), jnp.int32),
            pltpu.VMEM((CHUNK,), jnp.int32),
            pltpu.VMEM((CHUNK,), jnp.int32),
            pltpu.VMEM((CHUNK, EMB), jnp.float32),
            pltpu.VMEM((CHUNK, EMB), jnp.float32),
            pltpu.VMEM((CHUNK, EMB), jnp.float32),
            pltpu.VMEM((1, EMB), jnp.float32),
            pltpu.VMEM((CHUNK,), jnp.float32),
            pltpu.VMEM((CHUNK,), jnp.float32),
            pltpu.SemaphoreType.DMA,
        ],
    )
    return f(users, items, negs, umem, imem, vw)


def kernel(input_users, input_items, input_items_negative, user_memory,
           item_memory, v_w):
    pos, neg = _run(input_users.astype(jnp.int32),
                    input_items.astype(jnp.int32),
                    input_items_negative.astype(jnp.int32),
                    user_memory, item_memory, v_w)
    return pos.reshape(B, 1), neg.reshape(B, 1)


# speedup vs baseline: 2.4900x; 2.4900x over previous
import functools

import jax
import jax.numpy as jnp
from jax import lax
from jax.experimental import pallas as pl
from jax.experimental.pallas import tpu as pltpu
from jax.experimental.pallas import tpu_sc as plsc

B = 16384
EMB = 128
NC = 2
NS = 16
NW = NC * NS
BPW = B // NW
CHUNK = 256
NCHUNK = BPW // CHUNK


def _sc_body(users_hbm, items_hbm, negs_hbm, umem_hbm, imem_hbm, vw_hbm,
             pos_hbm, neg_hbm,
             uidx_v, iidx_v, nidx_v, urows_v, irows_v, nrows_v,
             v_v, pos_v, neg_v, sem):
    wid = lax.axis_index("s") * NC + lax.axis_index("c")
    base = wid * BPW
    pltpu.sync_copy(vw_hbm.at[0], v_v)

    for c in range(NCHUNK):
        cbase = base + c * CHUNK
        pltpu.sync_copy(users_hbm.at[pl.ds(cbase, CHUNK)], uidx_v)
        pltpu.sync_copy(items_hbm.at[pl.ds(cbase, CHUNK)], iidx_v)
        pltpu.sync_copy(negs_hbm.at[pl.ds(cbase, CHUNK)], nidx_v)
        cp_u = pltpu.async_copy(umem_hbm.at[uidx_v], urows_v, sem)
        cp_i = pltpu.async_copy(imem_hbm.at[iidx_v], irows_v, sem)
        cp_n = pltpu.async_copy(imem_hbm.at[nidx_v], nrows_v, sem)
        cp_u.wait()
        cp_i.wait()
        cp_n.wait()

        vj = [v_v[pl.ds(16 * j, 16)] for j in range(EMB // 16)]
        last_lane = jnp.arange(16, dtype=jnp.int32) == 15

        def group_body(g, carry):
            r0 = g * 16
            for rr in range(16):
                r = r0 + rr
                ap0 = ap1 = an0 = an1 = None
                for j in range(EMB // 16):
                    sl = pl.ds(16 * j, 16)
                    t = urows_v[r, sl] * vj[j]
                    p = t * irows_v[r, sl]
                    n = t * nrows_v[r, sl]
                    if j % 2 == 0:
                        ap0 = p if ap0 is None else ap0 + p
                        an0 = n if an0 is None else an0 + n
                    else:
                        ap1 = p if ap1 is None else ap1 + p
                        an1 = n if an1 is None else an1 + n
                sp = jnp.cumsum(ap0 + ap1)
                sn = jnp.cumsum(an0 + an1)
                ridx = jnp.full((16,), r, jnp.int32)
                plsc.store_scatter(pos_v, [ridx], jnp.maximum(sp, 0.0),
                                   mask=last_lane)
                plsc.store_scatter(neg_v, [ridx], jnp.maximum(sn, 0.0),
                                   mask=last_lane)
            return carry

        lax.fori_loop(0, CHUNK // 16, group_body, 0)
        pltpu.sync_copy(pos_v, pos_hbm.at[pl.ds(cbase, CHUNK)])
        pltpu.sync_copy(neg_v, neg_hbm.at[pl.ds(cbase, CHUNK)])


@jax.jit
def _run(users, items, negs, umem, imem, vw):
    f = pl.kernel(
        _sc_body,
        out_type=(
            jax.ShapeDtypeStruct((B,), jnp.float32),
            jax.ShapeDtypeStruct((B,), jnp.float32),
        ),
        mesh=plsc.VectorSubcoreMesh(core_axis_name="c", subcore_axis_name="s"),
        compiler_params=pltpu.CompilerParams(use_tc_tiling_on_sc=False,
                                             needs_layout_passes=False),
        scratch_types=[
            pltpu.VMEM((CHUNK,), jnp.int32),
            pltpu.VMEM((CHUNK,), jnp.int32),
            pltpu.VMEM((CHUNK,), jnp.int32),
            pltpu.VMEM((CHUNK, EMB), jnp.float32),
            pltpu.VMEM((CHUNK, EMB), jnp.float32),
            pltpu.VMEM((CHUNK, EMB), jnp.float32),
            pltpu.VMEM((EMB,), jnp.float32),
            pltpu.VMEM((CHUNK,), jnp.float32),
            pltpu.VMEM((CHUNK,), jnp.float32),
            pltpu.SemaphoreType.DMA,
        ],
    )
    return f(users, items, negs, umem, imem, vw)


def kernel(input_users, input_items, input_items_negative, user_memory,
           item_memory, v_w):
    pos, neg = _run(input_users.astype(jnp.int32),
                    input_items.astype(jnp.int32),
                    input_items_negative.astype(jnp.int32),
                    user_memory, item_memory, v_w)
    return pos.reshape(B, 1), neg.reshape(B, 1)


# speedup vs baseline: 2.9162x; 1.1712x over previous
import functools

import jax
import jax.numpy as jnp
from jax import lax
from jax.experimental import pallas as pl
from jax.experimental.pallas import tpu as pltpu
from jax.experimental.pallas import tpu_sc as plsc

B = 16384
EMB = 128
NC = 2
NS = 16
NW = NC * NS
BPW = B // NW
CHUNK = 256
NCHUNK = BPW // CHUNK


def _sc_body(users_hbm, items_hbm, negs_hbm, umem_hbm, imem_hbm, vw_hbm,
             pos_hbm, neg_hbm,
             uidx_v, iidx_v, nidx_v, urows_v, irows_v, nrows_v,
             v_v, pos_v, neg_v, stage_p, stage_n, sem):
    wid = lax.axis_index("s") * NC + lax.axis_index("c")
    base = wid * BPW
    pltpu.sync_copy(vw_hbm.at[0], v_v)

    for c in range(NCHUNK):
        cbase = base + c * CHUNK
        pltpu.sync_copy(users_hbm.at[pl.ds(cbase, CHUNK)], uidx_v)
        pltpu.sync_copy(items_hbm.at[pl.ds(cbase, CHUNK)], iidx_v)
        pltpu.sync_copy(negs_hbm.at[pl.ds(cbase, CHUNK)], nidx_v)
        cp_u = pltpu.async_copy(umem_hbm.at[uidx_v], urows_v, sem)
        cp_i = pltpu.async_copy(imem_hbm.at[iidx_v], irows_v, sem)
        cp_n = pltpu.async_copy(imem_hbm.at[nidx_v], nrows_v, sem)
        cp_u.wait()
        cp_i.wait()
        cp_n.wait()

        vj = [v_v[pl.ds(16 * j, 16)] for j in range(EMB // 16)]
        lane15 = 15 + 17 * lax.iota(jnp.int32, 16)

        def group_body(g, carry):
            r0 = g * 16

            def load_row(rr):
                r = r0 + rr
                sls = [pl.ds(16 * j, 16) for j in range(EMB // 16)]
                return ([urows_v[r, sl] for sl in sls],
                        [irows_v[r, sl] for sl in sls],
                        [nrows_v[r, sl] for sl in sls])

            def compute_row(ld, rr):
                us, is_, ns = ld
                ts = [us[j] * vj[j] for j in range(EMB // 16)]
                ap0 = ap1 = an0 = an1 = None
                for j in range(EMB // 16):
                    p = ts[j] * is_[j]
                    n = ts[j] * ns[j]
                    if j % 2 == 0:
                        ap0 = p if ap0 is None else ap0 + p
                        an0 = n if an0 is None else an0 + n
                    else:
                        ap1 = p if ap1 is None else ap1 + p
                        an1 = n if an1 is None else an1 + n
                stage_p[pl.ds(17 * rr, 16)] = jnp.cumsum(ap0 + ap1)
                stage_n[pl.ds(17 * rr, 16)] = jnp.cumsum(an0 + an1)

            cur = load_row(0)
            for rr in range(16):
                nxt = load_row(rr + 1) if rr < 15 else None
                compute_row(cur, rr)
                cur = nxt
            pg = plsc.load_gather(stage_p, [lane15])
            ng = plsc.load_gather(stage_n, [lane15])
            pos_v[pl.ds(r0, 16)] = jnp.maximum(pg, 0.0)
            neg_v[pl.ds(r0, 16)] = jnp.maximum(ng, 0.0)
            return carry

        lax.fori_loop(0, CHUNK // 16, group_body, 0)
        pltpu.sync_copy(pos_v, pos_hbm.at[pl.ds(cbase, CHUNK)])
        pltpu.sync_copy(neg_v, neg_hbm.at[pl.ds(cbase, CHUNK)])


@jax.jit
def _run(users, items, negs, umem, imem, vw):
    f = pl.kernel(
        _sc_body,
        out_type=(
            jax.ShapeDtypeStruct((B,), jnp.float32),
            jax.ShapeDtypeStruct((B,), jnp.float32),
        ),
        mesh=plsc.VectorSubcoreMesh(core_axis_name="c", subcore_axis_name="s"),
        compiler_params=pltpu.CompilerParams(use_tc_tiling_on_sc=False,
                                             needs_layout_passes=False),
        scratch_types=[
            pltpu.VMEM((CHUNK,), jnp.int32),
            pltpu.VMEM((CHUNK,), jnp.int32),
            pltpu.VMEM((CHUNK,), jnp.int32),
            pltpu.VMEM((CHUNK, EMB), jnp.float32),
            pltpu.VMEM((CHUNK, EMB), jnp.float32),
            pltpu.VMEM((CHUNK, EMB), jnp.float32),
            pltpu.VMEM((EMB,), jnp.float32),
            pltpu.VMEM((CHUNK,), jnp.float32),
            pltpu.VMEM((CHUNK,), jnp.float32),
            pltpu.VMEM((17 * 16,), jnp.float32),
            pltpu.VMEM((17 * 16,), jnp.float32),
            pltpu.SemaphoreType.DMA,
        ],
    )
    return f(users, items, negs, umem, imem, vw)


def kernel(input_users, input_items, input_items_negative, user_memory,
           item_memory, v_w):
    pos, neg = _run(input_users.astype(jnp.int32),
                    input_items.astype(jnp.int32),
                    input_items_negative.astype(jnp.int32),
                    user_memory, item_memory, v_w)
    return pos.reshape(B, 1), neg.reshape(B, 1)


# speedup vs baseline: 3.1065x; 1.0652x over previous
import jax
import jax.numpy as jnp
from jax import lax
from jax.experimental import pallas as pl
from jax.experimental.pallas import tpu as pltpu
from jax.experimental.pallas import tpu_sc as plsc

B = 16384
EMB = 128
NC = 2
NS = 16
NW = NC * NS
BPW = B // NW
CHUNK = 128
NCHUNK = BPW // CHUNK


def _sc_body(users_hbm, items_hbm, negs_hbm, umem_hbm, imem_hbm, vw_hbm,
             pos_hbm, neg_hbm,
             uidx_v, iidx_v, nidx_v,
             u_b0, i_b0, n_b0, u_b1, i_b1, n_b1,
             v_v, pos_v, neg_v, stage_p, stage_n, sem0, sem1):
    wid = lax.axis_index("s") * NC + lax.axis_index("c")
    base = wid * BPW
    pltpu.sync_copy(vw_hbm.at[0], v_v)
    pltpu.sync_copy(users_hbm.at[pl.ds(base, BPW)], uidx_v)
    pltpu.sync_copy(items_hbm.at[pl.ds(base, BPW)], iidx_v)
    pltpu.sync_copy(negs_hbm.at[pl.ds(base, BPW)], nidx_v)

    bufs = [(u_b0, i_b0, n_b0, sem0), (u_b1, i_b1, n_b1, sem1)]

    def start(c):
        ub, ib, nb, sem = bufs[c % 2]
        sl = pl.ds(c * CHUNK, CHUNK)
        return (pltpu.async_copy(umem_hbm.at[uidx_v.at[sl]], ub, sem),
                pltpu.async_copy(imem_hbm.at[iidx_v.at[sl]], ib, sem),
                pltpu.async_copy(imem_hbm.at[nidx_v.at[sl]], nb, sem))

    vj = [v_v[pl.ds(16 * j, 16)] for j in range(EMB // 16)]
    lane15 = 15 + 17 * lax.iota(jnp.int32, 16)

    def compute_chunk(c):
        urows_v, irows_v, nrows_v, _ = bufs[c % 2]

        def group_body(g, carry):
            r0 = g * 16

            def load_row(rr):
                r = r0 + rr
                sls = [pl.ds(16 * j, 16) for j in range(EMB // 16)]
                return ([urows_v[r, sl] for sl in sls],
                        [irows_v[r, sl] for sl in sls],
                        [nrows_v[r, sl] for sl in sls])

            def compute_row(ld, rr):
                us, is_, ns = ld
                ts = [us[j] * vj[j] for j in range(EMB // 16)]
                ap0 = ap1 = an0 = an1 = None
                for j in range(EMB // 16):
                    p = ts[j] * is_[j]
                    n = ts[j] * ns[j]
                    if j % 2 == 0:
                        ap0 = p if ap0 is None else ap0 + p
                        an0 = n if an0 is None else an0 + n
                    else:
                        ap1 = p if ap1 is None else ap1 + p
                        an1 = n if an1 is None else an1 + n
                stage_p[pl.ds(17 * rr, 16)] = jnp.cumsum(ap0 + ap1)
                stage_n[pl.ds(17 * rr, 16)] = jnp.cumsum(an0 + an1)

            cur = load_row(0)
            for rr in range(16):
                nxt = load_row(rr + 1) if rr < 15 else None
                compute_row(cur, rr)
                cur = nxt
            pg = plsc.load_gather(stage_p, [lane15])
            ng = plsc.load_gather(stage_n, [lane15])
            out = c * CHUNK + r0
            pos_v[pl.ds(out, 16)] = jnp.maximum(pg, 0.0)
            neg_v[pl.ds(out, 16)] = jnp.maximum(ng, 0.0)
            return carry

        lax.fori_loop(0, CHUNK // 16, group_body, 0)

    pending = {0: start(0)}
    for c in range(NCHUNK):
        if c + 1 < NCHUNK:
            pending[c + 1] = start(c + 1)
        for cp in pending.pop(c):
            cp.wait()
        compute_chunk(c)

    pltpu.sync_copy(pos_v, pos_hbm.at[pl.ds(base, BPW)])
    pltpu.sync_copy(neg_v, neg_hbm.at[pl.ds(base, BPW)])


@jax.jit
def _run(users, items, negs, umem, imem, vw):
    f = pl.kernel(
        _sc_body,
        out_type=(
            jax.ShapeDtypeStruct((B,), jnp.float32),
            jax.ShapeDtypeStruct((B,), jnp.float32),
        ),
        mesh=plsc.VectorSubcoreMesh(core_axis_name="c", subcore_axis_name="s"),
        compiler_params=pltpu.CompilerParams(use_tc_tiling_on_sc=False,
                                             needs_layout_passes=False),
        scratch_types=[
            pltpu.VMEM((BPW,), jnp.int32),
            pltpu.VMEM((BPW,), jnp.int32),
            pltpu.VMEM((BPW,), jnp.int32),
            pltpu.VMEM((CHUNK, EMB), jnp.float32),
            pltpu.VMEM((CHUNK, EMB), jnp.float32),
            pltpu.VMEM((CHUNK, EMB), jnp.float32),
            pltpu.VMEM((CHUNK, EMB), jnp.float32),
            pltpu.VMEM((CHUNK, EMB), jnp.float32),
            pltpu.VMEM((CHUNK, EMB), jnp.float32),
            pltpu.VMEM((EMB,), jnp.float32),
            pltpu.VMEM((BPW,), jnp.float32),
            pltpu.VMEM((BPW,), jnp.float32),
            pltpu.VMEM((17 * 16,), jnp.float32),
            pltpu.VMEM((17 * 16,), jnp.float32),
            pltpu.SemaphoreType.DMA,
            pltpu.SemaphoreType.DMA,
        ],
    )
    return f(users, items, negs, umem, imem, vw)


def kernel(input_users, input_items, input_items_negative, user_memory,
           item_memory, v_w):
    pos, neg = _run(input_users.astype(jnp.int32),
                    input_items.astype(jnp.int32),
                    input_items_negative.astype(jnp.int32),
                    user_memory, item_memory, v_w)
    return pos.reshape(B, 1), neg.reshape(B, 1)


# speedup vs baseline: 3.2686x; 1.0522x over previous
import jax
import jax.numpy as jnp
from jax import lax
from jax.experimental import pallas as pl
from jax.experimental.pallas import tpu as pltpu
from jax.experimental.pallas import tpu_sc as plsc

B = 16384
EMB = 128
NC = 2
NS = 16
NW = NC * NS
BPW = B // NW
CHUNK = 128
NCHUNK = BPW // CHUNK


def _sc_body(users_hbm, items_hbm, negs_hbm, umem_hbm, imem_hbm, vw_hbm,
             pos_hbm, neg_hbm,
             uidx_v, iidx_v, nidx_v,
             u_b0, i_b0, n_b0, u_b1, i_b1, n_b1,
             v_v, pos_v, neg_v, stage_p, stage_n, sem0, sem1):
    wid = lax.axis_index("s") * NC + lax.axis_index("c")
    base = wid * BPW
    pltpu.sync_copy(vw_hbm.at[0], v_v)
    pltpu.sync_copy(users_hbm.at[pl.ds(base, BPW)], uidx_v)
    pltpu.sync_copy(items_hbm.at[pl.ds(base, BPW)], iidx_v)
    pltpu.sync_copy(negs_hbm.at[pl.ds(base, BPW)], nidx_v)

    bufs = [(u_b0, i_b0, n_b0, sem0), (u_b1, i_b1, n_b1, sem1)]

    def start(c, parity):
        ub, ib, nb, sem = bufs[parity]
        sl = pl.ds(c * CHUNK, CHUNK)
        pltpu.async_copy(umem_hbm.at[uidx_v.at[sl]], ub, sem)
        pltpu.async_copy(imem_hbm.at[iidx_v.at[sl]], ib, sem)
        pltpu.async_copy(imem_hbm.at[nidx_v.at[sl]], nb, sem)

    def drain(parity):
        ub, ib, nb, sem = bufs[parity]
        dummy = umem_hbm.at[pl.ds(0, CHUNK)]
        pltpu.make_async_copy(dummy, ub, sem).wait()
        pltpu.make_async_copy(dummy, ib, sem).wait()
        pltpu.make_async_copy(dummy, nb, sem).wait()

    vj = [v_v[pl.ds(16 * j, 16)] for j in range(EMB // 16)]
    lane15 = 15 + 17 * lax.iota(jnp.int32, 16)

    def compute_chunk(c, parity):
        urows_v, irows_v, nrows_v, _ = bufs[parity]

        def group_body(g, carry):
            r0 = g * 16

            def load_row(rr):
                r = r0 + rr
                sls = [pl.ds(16 * j, 16) for j in range(EMB // 16)]
                return ([urows_v[r, sl] for sl in sls],
                        [irows_v[r, sl] for sl in sls],
                        [nrows_v[r, sl] for sl in sls])

            def compute_row(ld, rr):
                us, is_, ns = ld
                ts = [us[j] * vj[j] for j in range(EMB // 16)]
                ap0 = ap1 = an0 = an1 = None
                for j in range(EMB // 16):
                    p = ts[j] * is_[j]
                    n = ts[j] * ns[j]
                    if j % 2 == 0:
                        ap0 = p if ap0 is None else ap0 + p
                        an0 = n if an0 is None else an0 + n
                    else:
                        ap1 = p if ap1 is None else ap1 + p
                        an1 = n if an1 is None else an1 + n
                stage_p[pl.ds(17 * rr, 16)] = jnp.cumsum(ap0 + ap1)
                stage_n[pl.ds(17 * rr, 16)] = jnp.cumsum(an0 + an1)

            cur = load_row(0)
            for rr in range(16):
                nxt = load_row(rr + 1) if rr < 15 else None
                compute_row(cur, rr)
                cur = nxt
            pg = plsc.load_gather(stage_p, [lane15])
            ng = plsc.load_gather(stage_n, [lane15])
            out = c * CHUNK + r0
            pos_v[pl.ds(out, 16)] = jnp.maximum(pg, 0.0)
            neg_v[pl.ds(out, 16)] = jnp.maximum(ng, 0.0)
            return carry

        lax.fori_loop(0, CHUNK // 16, group_body, 0)

    start(0, 0)
    start(1, 1)

    def pair_body(p, carry):
        c0 = 2 * p
        for parity in range(2):
            c = c0 + parity
            drain(parity)
            compute_chunk(c, parity)

            @pl.when(c + 2 < NCHUNK)
            def _():
                start(c + 2, parity)

        return carry

    lax.fori_loop(0, NCHUNK // 2, pair_body, 0)

    pltpu.sync_copy(pos_v, pos_hbm.at[pl.ds(base, BPW)])
    pltpu.sync_copy(neg_v, neg_hbm.at[pl.ds(base, BPW)])


@jax.jit
def _run(users, items, negs, umem, imem, vw):
    f = pl.kernel(
        _sc_body,
        out_type=(
            jax.ShapeDtypeStruct((B,), jnp.float32),
            jax.ShapeDtypeStruct((B,), jnp.float32),
        ),
        mesh=plsc.VectorSubcoreMesh(core_axis_name="c", subcore_axis_name="s"),
        compiler_params=pltpu.CompilerParams(use_tc_tiling_on_sc=False,
                                             needs_layout_passes=False),
        scratch_types=[
            pltpu.VMEM((BPW,), jnp.int32),
            pltpu.VMEM((BPW,), jnp.int32),
            pltpu.VMEM((BPW,), jnp.int32),
            pltpu.VMEM((CHUNK, EMB), jnp.float32),
            pltpu.VMEM((CHUNK, EMB), jnp.float32),
            pltpu.VMEM((CHUNK, EMB), jnp.float32),
            pltpu.VMEM((CHUNK, EMB), jnp.float32),
            pltpu.VMEM((CHUNK, EMB), jnp.float32),
            pltpu.VMEM((CHUNK, EMB), jnp.float32),
            pltpu.VMEM((EMB,), jnp.float32),
            pltpu.VMEM((BPW,), jnp.float32),
            pltpu.VMEM((BPW,), jnp.float32),
            pltpu.VMEM((17 * 16,), jnp.float32),
            pltpu.VMEM((17 * 16,), jnp.float32),
            pltpu.SemaphoreType.DMA,
            pltpu.SemaphoreType.DMA,
        ],
    )
    return f(users, items, negs, umem, imem, vw)


def kernel(input_users, input_items, input_items_negative, user_memory,
           item_memory, v_w):
    pos, neg = _run(input_users.astype(jnp.int32),
                    input_items.astype(jnp.int32),
                    input_items_negative.astype(jnp.int32),
                    user_memory, item_memory, v_w)
    return pos.reshape(B, 1), neg.reshape(B, 1)


# speedup vs baseline: 3.4074x; 1.0425x over previous
import jax
import jax.numpy as jnp
from jax import lax
from jax.experimental import pallas as pl
from jax.experimental.pallas import tpu as pltpu
from jax.experimental.pallas import tpu_sc as plsc

B = 16384
EMB = 128
NC = 2
NS = 16
NW = NC * NS
BPW = B // NW
CHUNK = 128
NCHUNK = BPW // CHUNK


def _sc_body(users_hbm, items_hbm, negs_hbm, umem_hbm, imem_hbm, vw_hbm,
             pos_hbm, neg_hbm,
             uidx_v, iidx_v, nidx_v,
             u_b0, i_b0, n_b0, u_b1, i_b1, n_b1,
             v_v, pos_v, neg_v, sem0, sem1):
    wid = lax.axis_index("s") * NC + lax.axis_index("c")
    base = wid * BPW
    pltpu.sync_copy(vw_hbm.at[0], v_v)
    pltpu.sync_copy(users_hbm.at[pl.ds(base, BPW)], uidx_v)
    pltpu.sync_copy(items_hbm.at[pl.ds(base, BPW)], iidx_v)
    pltpu.sync_copy(negs_hbm.at[pl.ds(base, BPW)], nidx_v)

    bufs = [(u_b0, i_b0, n_b0, sem0), (u_b1, i_b1, n_b1, sem1)]

    def start(c, parity):
        ub, ib, nb, sem = bufs[parity]
        sl = pl.ds(c * CHUNK, CHUNK)
        pltpu.async_copy(umem_hbm.at[uidx_v.at[sl]], ub, sem)
        pltpu.async_copy(imem_hbm.at[iidx_v.at[sl]], ib, sem)
        pltpu.async_copy(imem_hbm.at[nidx_v.at[sl]], nb, sem)

    def drain(parity):
        ub, ib, nb, sem = bufs[parity]
        dummy = umem_hbm.at[pl.ds(0, CHUNK)]
        pltpu.make_async_copy(dummy, ub, sem).wait()
        pltpu.make_async_copy(dummy, ib, sem).wait()
        pltpu.make_async_copy(dummy, nb, sem).wait()

    vj = [v_v[pl.ds(16 * j, 16)] for j in range(EMB // 16)]
    last_lane = lax.iota(jnp.int32, 16) == 15

    def compute_chunk(c, parity):
        urows_v, irows_v, nrows_v, _ = bufs[parity]
        outbase = c * CHUNK

        @plsc.parallel_loop(0, CHUNK, 1, unroll=4)
        def row_body(r):
            sls = [pl.ds(16 * j, 16) for j in range(EMB // 16)]
            us = [urows_v[r, sl] for sl in sls]
            is_ = [irows_v[r, sl] for sl in sls]
            ns = [nrows_v[r, sl] for sl in sls]
            ts = [us[j] * vj[j] for j in range(EMB // 16)]
            ap0 = ap1 = an0 = an1 = None
            for j in range(EMB // 16):
                p = ts[j] * is_[j]
                n = ts[j] * ns[j]
                if j % 2 == 0:
                    ap0 = p if ap0 is None else ap0 + p
                    an0 = n if an0 is None else an0 + n
                else:
                    ap1 = p if ap1 is None else ap1 + p
                    an1 = n if an1 is None else an1 + n
            sp = jnp.cumsum(ap0 + ap1)
            sn = jnp.cumsum(an0 + an1)
            ridx = jnp.full((16,), outbase + r, jnp.int32)
            plsc.store_scatter(pos_v, [ridx], jnp.maximum(sp, 0.0),
                               mask=last_lane)
            plsc.store_scatter(neg_v, [ridx], jnp.maximum(sn, 0.0),
                               mask=last_lane)

    start(0, 0)
    start(1, 1)

    def pair_body(p, carry):
        c0 = 2 * p
        for parity in range(2):
            c = c0 + parity
            drain(parity)
            compute_chunk(c, parity)

            @pl.when(c + 2 < NCHUNK)
            def _():
                start(c + 2, parity)

        return carry

    lax.fori_loop(0, NCHUNK // 2, pair_body, 0)

    pltpu.sync_copy(pos_v, pos_hbm.at[pl.ds(base, BPW)])
    pltpu.sync_copy(neg_v, neg_hbm.at[pl.ds(base, BPW)])


@jax.jit
def _run(users, items, negs, umem, imem, vw):
    f = pl.kernel(
        _sc_body,
        out_type=(
            jax.ShapeDtypeStruct((B,), jnp.float32),
            jax.ShapeDtypeStruct((B,), jnp.float32),
        ),
        mesh=plsc.VectorSubcoreMesh(core_axis_name="c", subcore_axis_name="s"),
        compiler_params=pltpu.CompilerParams(use_tc_tiling_on_sc=False,
                                             needs_layout_passes=False),
        scratch_types=[
            pltpu.VMEM((BPW,), jnp.int32),
            pltpu.VMEM((BPW,), jnp.int32),
            pltpu.VMEM((BPW,), jnp.int32),
            pltpu.VMEM((CHUNK, EMB), jnp.float32),
            pltpu.VMEM((CHUNK, EMB), jnp.float32),
            pltpu.VMEM((CHUNK, EMB), jnp.float32),
            pltpu.VMEM((CHUNK, EMB), jnp.float32),
            pltpu.VMEM((CHUNK, EMB), jnp.float32),
            pltpu.VMEM((CHUNK, EMB), jnp.float32),
            pltpu.VMEM((EMB,), jnp.float32),
            pltpu.VMEM((BPW,), jnp.float32),
            pltpu.VMEM((BPW,), jnp.float32),
            pltpu.SemaphoreType.DMA,
            pltpu.SemaphoreType.DMA,
        ],
    )
    return f(users, items, negs, umem, imem, vw)


def kernel(input_users, input_items, input_items_negative, user_memory,
           item_memory, v_w):
    pos, neg = _run(input_users.astype(jnp.int32),
                    input_items.astype(jnp.int32),
                    input_items_negative.astype(jnp.int32),
                    user_memory, item_memory, v_w)
    return pos.reshape(B, 1), neg.reshape(B, 1)
